# baseline (speedup 1.0000x reference)
import functools

import jax
import jax.numpy as jnp
from jax import lax
from jax.experimental import pallas as pl
from jax.experimental.pallas import tpu as pltpu
from jax.experimental.pallas import tpu_sc as plsc

N = 10000
E = 160000
D = 256
ED = 16
L = 3

NCORES = 2
NSUB = 16
NW = NCORES * NSUB
CA = 128
CB = 64
E_PAD = 163840
N_ACC = 10240
ROWS_PER_TILE = N_ACC // NSUB
PREC = jax.lax.Precision.DEFAULT
F32 = jnp.float32
BF16 = jnp.bfloat16


def _pack_bf16_words(x):
    hw = x.shape[-1] // 2
    u = lax.bitcast_convert_type(x, jnp.uint32)
    r = (u + 0x7FFF + ((u >> 16) & 1)) >> 16
    word = r[..., :hw] | (r[..., hw:] << 16)
    return lax.bitcast_convert_type(word, jnp.int32)


def _edge_dense_body(ea_ref, we1_ref, be1_ref, we2_ref, be2_ref, a3_ref,
                     ba1_ref, q_ref, ef_ref):
    ea = ea_ref[...]
    we1 = we1_ref[0]
    he = jax.nn.relu(
        lax.dot_general(ea, we1, (((1,), (1,)), ((), ())), precision=PREC)
        + be1_ref[0])
    ef = (lax.dot_general(he, we2_ref[0], (((1,), (1,)), ((), ())),
                          precision=PREC) + be2_ref[0])
    q = (lax.dot_general(ea, a3_ref[0], (((1,), (1,)), ((), ())),
                         precision=PREC) + ba1_ref[0])
    q_ref[0] = _pack_bf16_words(q)
    n2 = ef.shape[0] // 2
    ef_ref[0, 0] = _pack_bf16_words(ef[:, :128].reshape(n2, 256))
    ef_ref[0, 1] = _pack_bf16_words(ef[:, 128:].reshape(n2, 256))


def _edge_dense(ea_p, We1, be1, We2, be2, A3, ba1):
    BE = 2048
    grid = (L, E_PAD // BE)
    return pl.pallas_call(
        _edge_dense_body,
        grid=grid,
        in_specs=[
            pl.BlockSpec((BE, ED), lambda l, e: (e, 0)),
            pl.BlockSpec((1, D, ED), lambda l, e: (l, 0, 0)),
            pl.BlockSpec((1, 1, D), lambda l, e: (l, 0, 0)),
            pl.BlockSpec((1, D, D), lambda l, e: (l, 0, 0)),
            pl.BlockSpec((1, 1, D), lambda l, e: (l, 0, 0)),
            pl.BlockSpec((1, D, ED), lambda l, e: (l, 0, 0)),
            pl.BlockSpec((1, 1, D), lambda l, e: (l, 0, 0)),
        ],
        out_specs=[
            pl.BlockSpec((1, BE, 128), lambda l, e: (l, e, 0)),
            pl.BlockSpec((1, 2, BE // 2, 128), lambda l, e: (l, 0, e, 0)),
        ],
        out_shape=[
            jax.ShapeDtypeStruct((L, E_PAD, 128), jnp.int32),
            jax.ShapeDtypeStruct((L, 2, E_PAD // 2, 128), jnp.int32),
        ],
    )(ea_p, We1, be1, We2, be2, A3, ba1)


def _node_dense_body(h_ref, wn_ref, a1_ref, a2_ref, p1_ref, p2_ref, xl2_ref):
    h = h_ref[...]
    xl = lax.dot_general(h, wn_ref[...], (((1,), (1,)), ((), ())),
                         precision=PREC)
    p1 = lax.dot_general(xl, a1_ref[...], (((1,), (1,)), ((), ())),
                         precision=PREC)
    p2 = lax.dot_general(xl, a2_ref[...], (((1,), (1,)), ((), ())),
                         precision=PREC)
    p1_ref[...] = _pack_bf16_words(p1)
    p2_ref[...] = _pack_bf16_words(p2)
    xl2_ref[0] = xl[:, :128]
    xl2_ref[1] = xl[:, 128:]


def _node_dense(h, Wn, A1, A2):
    BN = 1000
    grid = (N // BN,)
    return pl.pallas_call(
        _node_dense_body,
        grid=grid,
        in_specs=[
            pl.BlockSpec((BN, D), lambda i: (i, 0)),
            pl.BlockSpec((D, D), lambda i: (0, 0)),
            pl.BlockSpec((D, D), lambda i: (0, 0)),
            pl.BlockSpec((D, D), lambda i: (0, 0)),
        ],
        out_specs=[
            pl.BlockSpec((BN, 128), lambda i: (i, 0)),
            pl.BlockSpec((BN, 128), lambda i: (i, 0)),
            pl.BlockSpec((2, BN, 128), lambda i: (0, i, 0)),
        ],
        out_shape=[
            jax.ShapeDtypeStruct((N, 128), jnp.int32),
            jax.ShapeDtypeStruct((N, 128), jnp.int32),
            jax.ShapeDtypeStruct((2, N, 128), F32),
        ],
    )(h, Wn, A1, A2)


def _combine_body(s0_ref, s1_ref, h_ref, cb_ref, g_ref, b_ref, out_ref, *,
                  relu):
    u = jnp.concatenate([s0_ref[...], s1_ref[...]], axis=1) + cb_ref[...]
    m = jnp.mean(u, axis=-1, keepdims=True)
    v = jnp.mean((u - m) ** 2, axis=-1, keepdims=True)
    y = g_ref[...] * (u - m) * lax.rsqrt(v + 1e-5) + b_ref[...]
    if relu:
        y = jnp.maximum(y, 0.0)
    out_ref[...] = y + h_ref[...]


def _combine(s0, s1, h, cb, g, b, relu):
    BN = 1000
    return pl.pallas_call(
        functools.partial(_combine_body, relu=relu),
        grid=(N // BN,),
        in_specs=[
            pl.BlockSpec((BN, 128), lambda i: (i, 0)),
            pl.BlockSpec((BN, 128), lambda i: (i, 0)),
            pl.BlockSpec((BN, D), lambda i: (i, 0)),
            pl.BlockSpec((1, D), lambda i: (0, 0)),
            pl.BlockSpec((1, D), lambda i: (0, 0)),
            pl.BlockSpec((1, D), lambda i: (0, 0)),
        ],
        out_specs=pl.BlockSpec((BN, D), lambda i: (i, 0)),
        out_shape=jax.ShapeDtypeStruct((N, D), F32),
    )(s0, s1, h, cb, g, b)


_MESH = plsc.VectorSubcoreMesh(core_axis_name="c", subcore_axis_name="s",
                               num_cores=NCORES, num_subcores=NSUB)
_SC_PARAMS = pltpu.CompilerParams(needs_layout_passes=False)


def _attn_body(p1_hbm, p2_hbm, q_hbm, src_hbm, dst_hbm, wa2_hbm, ba2_hbm,
               a_hbm, srcall, dstall, p1r, p2r, qr, zbuf, wa2v, ba2v,
               sem0, sem1):
    c = lax.axis_index("c")
    s = lax.axis_index("s")
    w = s * NCORES + c
    ncht = E_PAD // CA // NW
    base0 = w * (ncht * CA)
    sems = [sem0, sem1]

    pltpu.sync_copy(src_hbm.at[pl.ds(base0, ncht * CA)], srcall)
    pltpu.sync_copy(dst_hbm.at[pl.ds(base0, ncht * CA)], dstall)
    pltpu.sync_copy(wa2_hbm, wa2v)
    pltpu.sync_copy(ba2_hbm, ba2v)

    def issue(t, b):
        isl = pl.ds(t * CA, CA)
        pltpu.async_copy(p1_hbm.at[dstall.at[isl]], p1r.at[b], sems[b])
        pltpu.async_copy(p2_hbm.at[srcall.at[isl]], p2r.at[b], sems[b])
        pltpu.async_copy(q_hbm.at[pl.ds(base0 + t * CA, CA)], qr.at[b],
                         sems[b])

    def wait(b):
        isl = pl.ds(0, CA)
        pltpu.make_async_copy(p1_hbm.at[dstall.at[isl]], p1r.at[b],
                              sems[b]).wait()
        pltpu.make_async_copy(p2_hbm.at[srcall.at[isl]], p2r.at[b],
                              sems[b]).wait()
        pltpu.make_async_copy(q_hbm.at[pl.ds(base0, CA)], qr.at[b],
                              sems[b]).wait()

    issue(0, 0)
    wa2_regs = [plsc.bitcast(wa2v[pl.ds(16 * g, 16)], BF16)
                for g in range(8)]
    ba2 = ba2v[...]
    lane0 = lax.iota(jnp.int32, 16) == 0
    zeros16 = jnp.zeros((16,), F32)

    def group_body(g, carry):
        for b in range(2):
            t = g * 2 + b
            wait(b)
            issue(jnp.minimum(t + 1, ncht - 1), 1 - b)

            def edge_body(e, carry2):
                acc0 = zeros16
                acc1 = zeros16
                for j in range(8):
                    sl = pl.ds(16 * j, 16)
                    v = (plsc.bitcast(p1r[b, e, sl], BF16)
                         + plsc.bitcast(p2r[b, e, sl], BF16)
                         + plsc.bitcast(qr[b, e, sl], BF16))
                    v = jnp.maximum(v, 0.0)
                    prod = v * wa2_regs[j]
                    u0, u1 = plsc.unpack(
                        prod, format=plsc.PackFormat.INTERLEAVED)
                    acc0 = acc0 + u0
                    acc1 = acc1 + u1
                zvec = jnp.broadcast_to(jnp.sum(acc0 + acc1), (16,))
                eidx = jnp.broadcast_to(e.astype(jnp.int32), (16,))
                plsc.store_scatter(zbuf, [eidx], zvec, mask=lane0)
                return carry2

            lax.fori_loop(0, CA, edge_body, 0)
            for i in range(CA // 16):
                sl = pl.ds(16 * i, 16)
                z = zbuf[sl] + ba2
                zbuf[sl] = 1.0 / (1.0 + jnp.exp(-z))
            pltpu.sync_copy(zbuf, a_hbm.at[pl.ds(base0 + t * CA, CA)])
        return carry

    lax.fori_loop(0, ncht // 2, group_body, 0)
    wait(0)


def _attn(p1b, p2b, q_l, src_p, dst_p, wa2, ba2v):
    f = pl.kernel(
        _attn_body,
        out_type=jax.ShapeDtypeStruct((E_PAD,), F32),
        mesh=_MESH,
        compiler_params=_SC_PARAMS,
        scratch_types=[
            pltpu.VMEM((E_PAD // NW,), jnp.int32),
            pltpu.VMEM((E_PAD // NW,), jnp.int32),
            pltpu.VMEM((2, CA, 128), jnp.int32),
            pltpu.VMEM((2, CA, 128), jnp.int32),
            pltpu.VMEM((2, CA, 128), jnp.int32),
            pltpu.VMEM((CA,), F32),
            pltpu.VMEM((128,), jnp.int32),
            pltpu.VMEM((16,), F32),
            pltpu.SemaphoreType.DMA,
            pltpu.SemaphoreType.DMA,
        ],
    )
    return f(p1b, p2b, q_l, src_p, dst_p, wa2, ba2v)


def _agg_body(xl2_hbm, ef2_hbm, a_hbm, src_hbm, dst_hbm,
              s_hbm, src2all, dstv, av, xlr, efr, acc,
              sem0, sem1, scs0, scs1):
    c = lax.axis_index("c")
    s = lax.axis_index("s")
    ncht = E_PAD // CB // NSUB
    base0 = s * (ncht * CB)
    sems = [sem0, sem1]
    scs = [scs0, scs1]

    pltpu.sync_copy(src_hbm.at[pl.ds(base0, ncht * CB)], src2all)
    off = c * N

    def addoff(i, carry):
        sl = pl.ds(16 * i, 16)
        src2all[sl] = src2all[sl] + off
        return carry

    lax.fori_loop(0, ncht * CB // 16, addoff, 0)

    zeros16 = jnp.zeros((16,), F32)

    def zrow(r, carry):
        for j in range(8):
            xlr[0, r, pl.ds(16 * j, 16)] = zeros16
        return carry

    lax.fori_loop(0, CB, zrow, 0)
    r0 = s * ROWS_PER_TILE
    for j in range(ROWS_PER_TILE // CB):
        pltpu.sync_copy(xlr.at[0], acc.at[pl.ds(r0 + j * CB, CB)])
    plsc.subcore_barrier()

    def issue(t, b):
        pltpu.async_copy(xl2_hbm.at[src2all.at[pl.ds(t * CB, CB)]],
                         xlr.at[b], sems[b])
        pltpu.async_copy(
            ef2_hbm.at[pl.ds(c * (E_PAD // 2) + s * (ncht * CB // 2)
                             + t * (CB // 2), CB // 2)],
            efr.at[b], sems[b])
        pltpu.async_copy(dst_hbm.at[pl.ds(base0 + t * CB, CB)],
                         dstv.at[b], sems[b])
        pltpu.async_copy(a_hbm.at[pl.ds(base0 + t * CB, CB)],
                         av.at[b, pl.ds(0, CB)], sems[b])

    def wait(b):
        pltpu.make_async_copy(xl2_hbm.at[src2all.at[pl.ds(0, CB)]],
                              xlr.at[b], sems[b]).wait()
        pltpu.make_async_copy(ef2_hbm.at[pl.ds(0, CB // 2)], efr.at[b],
                              sems[b]).wait()
        pltpu.make_async_copy(dst_hbm.at[pl.ds(0, CB)], dstv.at[b],
                              sems[b]).wait()
        pltpu.make_async_copy(a_hbm.at[pl.ds(0, CB)],
                              av.at[b, pl.ds(0, CB)], sems[b]).wait()

    def wait_scatter(b):
        pltpu.make_async_copy(xlr.at[b], acc.at[dstv.at[b]], scs[b]).wait()

    issue(0, 0)

    def group_body(g, carry):
        for b in range(2):
            t = g * 2 + b
            wait(b)
            if b == 0:
                @pl.when(g >= 1)
                def _():
                    wait_scatter(1)
            else:
                wait_scatter(0)
            issue(jnp.minimum(t + 1, ncht - 1), 1 - b)

            def pair_body(p, carry2):
                e0 = 2 * p
                ae0 = jnp.broadcast_to(av[b, pl.ds(e0, 16)][0], (16,))
                ae1 = jnp.broadcast_to(av[b, pl.ds(e0 + 1, 16)][0], (16,))
                for j in range(8):
                    sl = pl.ds(16 * j, 16)
                    u0, u1 = plsc.unpack(
                        plsc.bitcast(efr[b, p, sl], BF16),
                        format=plsc.PackFormat.INTERLEAVED)
                    xlr[b, e0, sl] = (xlr[b, e0, sl] + u0) * ae0
                    xlr[b, e0 + 1, sl] = (xlr[b, e0 + 1, sl] + u1) * ae1
                return carry2

            lax.fori_loop(0, CB // 2, pair_body, 0)
            pltpu.async_copy(xlr.at[b], acc.at[dstv.at[b]], scs[b],
                             add=True)
        return carry

    lax.fori_loop(0, ncht // 2, group_body, 0)
    wait(0)
    wait_scatter(1)
    plsc.subcore_barrier()

    for j in range(ROWS_PER_TILE // 128):
        pltpu.sync_copy(acc.at[pl.ds(r0 + j * 128, 128)],
                        s_hbm.at[pl.ds(c * N_ACC + r0 + j * 128, 128)])


def _aggregate(xl2f, ef2f, a, src_p, dst_p):
    f = pl.kernel(
        _agg_body,
        out_type=jax.ShapeDtypeStruct((NCORES * N_ACC, 128), F32),
        mesh=_MESH,
        compiler_params=_SC_PARAMS,
        scratch_types=[
            pltpu.VMEM((E_PAD // NSUB,), jnp.int32),
            pltpu.VMEM((2, CB), jnp.int32),
            pltpu.VMEM((2, CB + 16), F32),
            pltpu.VMEM((2, CB, 128), F32),
            pltpu.VMEM((2, CB // 2, 128), jnp.int32),
            pltpu.VMEM_SHARED((N_ACC, 128), F32),
            pltpu.SemaphoreType.DMA,
            pltpu.SemaphoreType.DMA,
            pltpu.SemaphoreType.DMA,
            pltpu.SemaphoreType.DMA,
        ],
    )
    return f(xl2f, ef2f, a, src_p, dst_p)


def kernel(x, edge_index, edge_attr, W_node, W_e1, b_e1, W_e2, b_e2,
           W_a1, b_a1, W_a2, b_a2, conv_bias, ln_g, ln_b):
    src = edge_index[0].astype(jnp.int32)
    dst = edge_index[1].astype(jnp.int32)
    pad = E_PAD - E
    src_p = jnp.concatenate([src, jnp.zeros((pad,), jnp.int32)])
    dst_p = jnp.concatenate([dst, jnp.full((pad,), N, jnp.int32)])
    ea_p = jnp.pad(edge_attr, ((0, pad), (0, 0)))

    A1 = W_a1[:, :, :D]
    A2 = W_a1[:, :, D:2 * D]
    A3 = W_a1[:, :, 2 * D:]

    q_all, ef_all = _edge_dense(ea_p, W_e1, b_e1.reshape(L, 1, D), W_e2,
                                b_e2.reshape(L, 1, D), A3,
                                b_a1.reshape(L, 1, D))

    h = x
    for i in range(L):
        p1b, p2b, xl2 = _node_dense(h, W_node[i], A1[i], A2[i])
        xl2f = xl2.reshape(2 * N, 128)
        wa2 = _pack_bf16_words(W_a2[i, 0])
        ba2v = jnp.full((16,), b_a2[i, 0], F32)
        a = _attn(p1b, p2b, q_all[i], src_p, dst_p, wa2, ba2v)
        ef2f = ef_all[i].reshape(E_PAD, 128)
        sflat = _aggregate(xl2f, ef2f, a, src_p, dst_p)
        s_split = sflat.reshape(2, N_ACC, 128)
        h = _combine(s_split[0, :N], s_split[1, :N], h, conv_bias[i:i + 1],
                     ln_g[i:i + 1], ln_b[i:i + 1], relu=(i < L - 1))
    return h


# speedup vs baseline: 1.2094x; 1.2094x over previous
import functools

import jax
import jax.numpy as jnp
from jax import lax
from jax.experimental import pallas as pl
from jax.experimental.pallas import tpu as pltpu
from jax.experimental.pallas import tpu_sc as plsc

N = 10000
E = 160000
D = 256
ED = 16
L = 3

NCORES = 2
NSUB = 16
NW = NCORES * NSUB
CA = 128
CB = 64
E_PAD = 163840
N_ACC = 10240
ROWS_PER_TILE = N_ACC // NSUB
PREC = jax.lax.Precision.DEFAULT
F32 = jnp.float32
BF16 = jnp.bfloat16


def _pack_bf16_words(x):
    hw = x.shape[-1] // 2
    u = lax.bitcast_convert_type(x, jnp.uint32)
    r = (u + 0x7FFF + ((u >> 16) & 1)) >> 16
    word = r[..., :hw] | (r[..., hw:] << 16)
    return lax.bitcast_convert_type(word, jnp.int32)


def _edge_dense_body(ea_ref, we1_ref, be1_ref, we2_ref, be2_ref, a3_ref,
                     ba1_ref, q_ref, ef_ref):
    ea = ea_ref[...]
    we1 = we1_ref[0]
    he = jax.nn.relu(
        lax.dot_general(ea, we1, (((1,), (1,)), ((), ())), precision=PREC)
        + be1_ref[0])
    ef = (lax.dot_general(he, we2_ref[0], (((1,), (1,)), ((), ())),
                          precision=PREC) + be2_ref[0])
    q = (lax.dot_general(ea, a3_ref[0], (((1,), (1,)), ((), ())),
                         precision=PREC) + ba1_ref[0])
    q_ref[0] = _pack_bf16_words(q)
    ef_ref[0, 0] = ef[:, :128]
    ef_ref[0, 1] = ef[:, 128:]


def _edge_dense(ea_p, We1, be1, We2, be2, A3, ba1):
    BE = 2048
    nl = We1.shape[0]
    grid = (nl, E_PAD // BE)
    return pl.pallas_call(
        _edge_dense_body,
        grid=grid,
        in_specs=[
            pl.BlockSpec((BE, ED), lambda l, e: (e, 0)),
            pl.BlockSpec((1, D, ED), lambda l, e: (l, 0, 0)),
            pl.BlockSpec((1, 1, D), lambda l, e: (l, 0, 0)),
            pl.BlockSpec((1, D, D), lambda l, e: (l, 0, 0)),
            pl.BlockSpec((1, 1, D), lambda l, e: (l, 0, 0)),
            pl.BlockSpec((1, D, ED), lambda l, e: (l, 0, 0)),
            pl.BlockSpec((1, 1, D), lambda l, e: (l, 0, 0)),
        ],
        out_specs=[
            pl.BlockSpec((1, BE, 128), lambda l, e: (l, e, 0)),
            pl.BlockSpec((1, 2, BE, 128), lambda l, e: (l, 0, e, 0)),
        ],
        out_shape=[
            jax.ShapeDtypeStruct((nl, E_PAD, 128), jnp.int32),
            jax.ShapeDtypeStruct((nl, 2, E_PAD, 128), F32),
        ],
    )(ea_p, We1, be1, We2, be2, A3, ba1)


def _node_dense_body(h_ref, wn_ref, a1_ref, a2_ref, p1_ref, p2_ref, xl2_ref):
    h = h_ref[...]
    xl = lax.dot_general(h, wn_ref[...], (((1,), (1,)), ((), ())),
                         precision=PREC)
    p1 = lax.dot_general(xl, a1_ref[...], (((1,), (1,)), ((), ())),
                         precision=PREC)
    p2 = lax.dot_general(xl, a2_ref[...], (((1,), (1,)), ((), ())),
                         precision=PREC)
    p1_ref[...] = _pack_bf16_words(p1)
    p2_ref[...] = _pack_bf16_words(p2)
    xl2_ref[0] = xl[:, :128]
    xl2_ref[1] = xl[:, 128:]


def _node_dense(h, Wn, A1, A2):
    BN = 1000
    grid = (N // BN,)
    return pl.pallas_call(
        _node_dense_body,
        grid=grid,
        in_specs=[
            pl.BlockSpec((BN, D), lambda i: (i, 0)),
            pl.BlockSpec((D, D), lambda i: (0, 0)),
            pl.BlockSpec((D, D), lambda i: (0, 0)),
            pl.BlockSpec((D, D), lambda i: (0, 0)),
        ],
        out_specs=[
            pl.BlockSpec((BN, 128), lambda i: (i, 0)),
            pl.BlockSpec((BN, 128), lambda i: (i, 0)),
            pl.BlockSpec((2, BN, 128), lambda i: (0, i, 0)),
        ],
        out_shape=[
            jax.ShapeDtypeStruct((N, 128), jnp.int32),
            jax.ShapeDtypeStruct((N, 128), jnp.int32),
            jax.ShapeDtypeStruct((2, N, 128), F32),
        ],
    )(h, Wn, A1, A2)


def _combine_body(s0_ref, s1_ref, h_ref, cb_ref, g_ref, b_ref, out_ref, *,
                  relu):
    u = jnp.concatenate([s0_ref[...], s1_ref[...]], axis=1) + cb_ref[...]
    m = jnp.mean(u, axis=-1, keepdims=True)
    v = jnp.mean((u - m) ** 2, axis=-1, keepdims=True)
    y = g_ref[...] * (u - m) * lax.rsqrt(v + 1e-5) + b_ref[...]
    if relu:
        y = jnp.maximum(y, 0.0)
    out_ref[...] = y + h_ref[...]


def _combine(s0, s1, h, cb, g, b, relu):
    BN = 1000
    return pl.pallas_call(
        functools.partial(_combine_body, relu=relu),
        grid=(N // BN,),
        in_specs=[
            pl.BlockSpec((BN, 128), lambda i: (i, 0)),
            pl.BlockSpec((BN, 128), lambda i: (i, 0)),
            pl.BlockSpec((BN, D), lambda i: (i, 0)),
            pl.BlockSpec((1, D), lambda i: (0, 0)),
            pl.BlockSpec((1, D), lambda i: (0, 0)),
            pl.BlockSpec((1, D), lambda i: (0, 0)),
        ],
        out_specs=pl.BlockSpec((BN, D), lambda i: (i, 0)),
        out_shape=jax.ShapeDtypeStruct((N, D), F32),
    )(s0, s1, h, cb, g, b)


_MESH = plsc.VectorSubcoreMesh(core_axis_name="c", subcore_axis_name="s",
                               num_cores=NCORES, num_subcores=NSUB)
_SC_PARAMS = pltpu.CompilerParams(needs_layout_passes=False)


def _attn_body(p1_hbm, p2_hbm, q_hbm, src_hbm, dst_hbm, wa2_hbm, ba2_hbm,
               a_hbm, srcall, dstall, p1r, p2r, qr, zbuf, wa2v, ba2v,
               sem0, sem1):
    c = lax.axis_index("c")
    s = lax.axis_index("s")
    w = s * NCORES + c
    ncht = E_PAD // CA // NW
    base0 = w * (ncht * CA)
    sems = [sem0, sem1]

    pltpu.sync_copy(src_hbm.at[pl.ds(base0, ncht * CA)], srcall)
    pltpu.sync_copy(dst_hbm.at[pl.ds(base0, ncht * CA)], dstall)
    pltpu.sync_copy(wa2_hbm, wa2v)
    pltpu.sync_copy(ba2_hbm, ba2v)

    def issue(t, b):
        isl = pl.ds(t * CA, CA)
        pltpu.async_copy(p1_hbm.at[dstall.at[isl]], p1r.at[b], sems[b])
        pltpu.async_copy(p2_hbm.at[srcall.at[isl]], p2r.at[b], sems[b])
        pltpu.async_copy(q_hbm.at[pl.ds(base0 + t * CA, CA)], qr.at[b],
                         sems[b])

    def wait(b):
        isl = pl.ds(0, CA)
        pltpu.make_async_copy(p1_hbm.at[dstall.at[isl]], p1r.at[b],
                              sems[b]).wait()
        pltpu.make_async_copy(p2_hbm.at[srcall.at[isl]], p2r.at[b],
                              sems[b]).wait()
        pltpu.make_async_copy(q_hbm.at[pl.ds(base0, CA)], qr.at[b],
                              sems[b]).wait()

    issue(0, 0)
    wa2_regs = [plsc.bitcast(wa2v[pl.ds(16 * g, 16)], BF16)
                for g in range(8)]
    ba2 = ba2v[...]
    lane0 = lax.iota(jnp.int32, 16) == 0
    zeros16 = jnp.zeros((16,), F32)

    def group_body(g, carry):
        for b in range(2):
            t = g * 2 + b
            wait(b)
            issue(jnp.minimum(t + 1, ncht - 1), 1 - b)

            def edge_body(e, carry2):
                acc0 = zeros16
                acc1 = zeros16
                for j in range(8):
                    sl = pl.ds(16 * j, 16)
                    v = (plsc.bitcast(p1r[b, e, sl], BF16)
                         + plsc.bitcast(p2r[b, e, sl], BF16)
                         + plsc.bitcast(qr[b, e, sl], BF16))
                    v = jnp.maximum(v, 0.0)
                    prod = v * wa2_regs[j]
                    u0, u1 = plsc.unpack(
                        prod, format=plsc.PackFormat.INTERLEAVED)
                    acc0 = acc0 + u0
                    acc1 = acc1 + u1
                zvec = jnp.broadcast_to(jnp.sum(acc0 + acc1), (16,))
                eidx = jnp.broadcast_to(e.astype(jnp.int32), (16,))
                plsc.store_scatter(zbuf, [eidx], zvec, mask=lane0)
                return carry2

            lax.fori_loop(0, CA, edge_body, 0)
            for i in range(CA // 16):
                sl = pl.ds(16 * i, 16)
                z = zbuf[sl] + ba2
                zbuf[sl] = 1.0 / (1.0 + jnp.exp(-z))
            pltpu.sync_copy(zbuf, a_hbm.at[pl.ds(base0 + t * CA, CA)])
        return carry

    lax.fori_loop(0, ncht // 2, group_body, 0)
    wait(0)


def _attn(p1b, p2b, q_l, src_p, dst_p, wa2, ba2v):
    f = pl.kernel(
        _attn_body,
        out_type=jax.ShapeDtypeStruct((E_PAD,), F32),
        mesh=_MESH,
        compiler_params=_SC_PARAMS,
        scratch_types=[
            pltpu.VMEM((E_PAD // NW,), jnp.int32),
            pltpu.VMEM((E_PAD // NW,), jnp.int32),
            pltpu.VMEM((2, CA, 128), jnp.int32),
            pltpu.VMEM((2, CA, 128), jnp.int32),
            pltpu.VMEM((2, CA, 128), jnp.int32),
            pltpu.VMEM((CA,), F32),
            pltpu.VMEM((128,), jnp.int32),
            pltpu.VMEM((16,), F32),
            pltpu.SemaphoreType.DMA,
            pltpu.SemaphoreType.DMA,
        ],
    )
    return f(p1b, p2b, q_l, src_p, dst_p, wa2, ba2v)


def _agg_body(xl2_hbm, ef2_hbm, a_hbm, src_hbm, dst_hbm,
              s_hbm, src2all, dstv, av, xlr, efr, acc,
              sem0, sem1, scs0, scs1):
    c = lax.axis_index("c")
    s = lax.axis_index("s")
    ncht = E_PAD // CB // NSUB
    base0 = s * (ncht * CB)
    sems = [sem0, sem1]
    scs = [scs0, scs1]

    pltpu.sync_copy(src_hbm.at[pl.ds(base0, ncht * CB)], src2all)
    off = c * N

    def addoff(i, carry):
        sl = pl.ds(16 * i, 16)
        src2all[sl] = src2all[sl] + off
        return carry

    lax.fori_loop(0, ncht * CB // 16, addoff, 0)

    zeros16 = jnp.zeros((16,), F32)

    def zrow(r, carry):
        for j in range(8):
            xlr[0, r, pl.ds(16 * j, 16)] = zeros16
        return carry

    lax.fori_loop(0, CB, zrow, 0)
    r0 = s * ROWS_PER_TILE
    for j in range(ROWS_PER_TILE // CB):
        pltpu.sync_copy(xlr.at[0], acc.at[pl.ds(r0 + j * CB, CB)])
    plsc.subcore_barrier()

    def issue(t, b):
        pltpu.async_copy(xl2_hbm.at[src2all.at[pl.ds(t * CB, CB)]],
                         xlr.at[b], sems[b])
        pltpu.async_copy(ef2_hbm.at[pl.ds(c * E_PAD + base0 + t * CB, CB)],
                         efr.at[b], sems[b])
        pltpu.async_copy(dst_hbm.at[pl.ds(base0 + t * CB, CB)],
                         dstv.at[b], sems[b])
        pltpu.async_copy(a_hbm.at[pl.ds(base0 + t * CB, CB)],
                         av.at[b, pl.ds(0, CB)], sems[b])

    def wait(b):
        pltpu.make_async_copy(xl2_hbm.at[src2all.at[pl.ds(0, CB)]],
                              xlr.at[b], sems[b]).wait()
        pltpu.make_async_copy(ef2_hbm.at[pl.ds(0, CB)], efr.at[b],
                              sems[b]).wait()
        pltpu.make_async_copy(dst_hbm.at[pl.ds(0, CB)], dstv.at[b],
                              sems[b]).wait()
        pltpu.make_async_copy(a_hbm.at[pl.ds(0, CB)],
                              av.at[b, pl.ds(0, CB)], sems[b]).wait()

    def wait_scatter(b):
        pltpu.make_async_copy(xlr.at[b], acc.at[dstv.at[b]], scs[b]).wait()

    issue(0, 0)

    def group_body(g, carry):
        for b in range(2):
            t = g * 2 + b
            wait(b)
            if b == 0:
                @pl.when(g >= 1)
                def _():
                    wait_scatter(1)
            else:
                wait_scatter(0)
            issue(jnp.minimum(t + 1, ncht - 1), 1 - b)

            def pair_body(p, carry2):
                e0 = 2 * p
                ae0 = jnp.broadcast_to(av[b, pl.ds(e0, 16)][0], (16,))
                ae1 = jnp.broadcast_to(av[b, pl.ds(e0 + 1, 16)][0], (16,))
                for j in range(8):
                    sl = pl.ds(16 * j, 16)
                    xlr[b, e0, sl] = (xlr[b, e0, sl] + efr[b, e0, sl]) * ae0
                    xlr[b, e0 + 1, sl] = (
                        xlr[b, e0 + 1, sl] + efr[b, e0 + 1, sl]) * ae1
                return carry2

            lax.fori_loop(0, CB // 2, pair_body, 0)
            pltpu.async_copy(xlr.at[b], acc.at[dstv.at[b]], scs[b],
                             add=True)
        return carry

    lax.fori_loop(0, ncht // 2, group_body, 0)
    wait(0)
    wait_scatter(1)
    plsc.subcore_barrier()

    for j in range(ROWS_PER_TILE // 128):
        pltpu.sync_copy(acc.at[pl.ds(r0 + j * 128, 128)],
                        s_hbm.at[pl.ds(c * N_ACC + r0 + j * 128, 128)])


def _aggregate(xl2f, ef2f, a, src_p, dst_p):
    f = pl.kernel(
        _agg_body,
        out_type=jax.ShapeDtypeStruct((NCORES * N_ACC, 128), F32),
        mesh=_MESH,
        compiler_params=_SC_PARAMS,
        scratch_types=[
            pltpu.VMEM((E_PAD // NSUB,), jnp.int32),
            pltpu.VMEM((2, CB), jnp.int32),
            pltpu.VMEM((2, CB + 16), F32),
            pltpu.VMEM((2, CB, 128), F32),
            pltpu.VMEM((2, CB, 128), F32),
            pltpu.VMEM_SHARED((N_ACC, 128), F32),
            pltpu.SemaphoreType.DMA,
            pltpu.SemaphoreType.DMA,
            pltpu.SemaphoreType.DMA,
            pltpu.SemaphoreType.DMA,
        ],
    )
    return f(xl2f, ef2f, a, src_p, dst_p)


def kernel(x, edge_index, edge_attr, W_node, W_e1, b_e1, W_e2, b_e2,
           W_a1, b_a1, W_a2, b_a2, conv_bias, ln_g, ln_b):
    src = edge_index[0].astype(jnp.int32)
    dst = edge_index[1].astype(jnp.int32)
    pad = E_PAD - E
    src_p = jnp.concatenate([src, jnp.zeros((pad,), jnp.int32)])
    dst_p = jnp.concatenate([dst, jnp.full((pad,), N, jnp.int32)])
    ea_p = jnp.pad(edge_attr, ((0, pad), (0, 0)))

    A1 = W_a1[:, :, :D]
    A2 = W_a1[:, :, D:2 * D]
    A3 = W_a1[:, :, 2 * D:]

    h = x
    for i in range(L):
        q_i, ef_i = _edge_dense(ea_p, W_e1[i:i + 1],
                                b_e1[i:i + 1].reshape(1, 1, D),
                                W_e2[i:i + 1], b_e2[i:i + 1].reshape(1, 1, D),
                                A3[i:i + 1], b_a1[i:i + 1].reshape(1, 1, D))
        p1b, p2b, xl2 = _node_dense(h, W_node[i], A1[i], A2[i])
        xl2f = xl2.reshape(2 * N, 128)
        wa2 = _pack_bf16_words(W_a2[i, 0])
        ba2v = jnp.full((16,), b_a2[i, 0], F32)
        a = _attn(p1b, p2b, q_i[0], src_p, dst_p, wa2, ba2v)
        ef2f = ef_i[0].reshape(2 * E_PAD, 128)
        sflat = _aggregate(xl2f, ef2f, a, src_p, dst_p)
        s_split = sflat.reshape(2, N_ACC, 128)
        h = _combine(s_split[0, :N], s_split[1, :N], h, conv_bias[i:i + 1],
                     ln_g[i:i + 1], ln_b[i:i + 1], relu=(i < L - 1))
    return h


# speedup vs baseline: 1.2133x; 1.0032x over previous
import functools

import jax
import jax.numpy as jnp
from jax import lax
from jax.experimental import pallas as pl
from jax.experimental.pallas import tpu as pltpu
from jax.experimental.pallas import tpu_sc as plsc

N = 10000
E = 160000
D = 256
ED = 16
L = 3

NCORES = 2
NSUB = 16
NW = NCORES * NSUB
CA = 128
CB = 64
E_PAD = 163840
N_ACC = 10240
ROWS_PER_TILE = N_ACC // NSUB
PREC = jax.lax.Precision.DEFAULT
F32 = jnp.float32
BF16 = jnp.bfloat16


def _pack_bf16_words(x):
    hw = x.shape[-1] // 2
    u = lax.bitcast_convert_type(x, jnp.uint32)
    r = (u + 0x7FFF + ((u >> 16) & 1)) >> 16
    word = r[..., :hw] | (r[..., hw:] << 16)
    return lax.bitcast_convert_type(word, jnp.int32)


def _edge_dense_body(ea_ref, we1_ref, be1_ref, we2_ref, be2_ref, a3_ref,
                     ba1_ref, q_ref, ef_ref):
    ea = ea_ref[...]
    we1 = we1_ref[0]
    he = jax.nn.relu(
        lax.dot_general(ea, we1, (((1,), (1,)), ((), ())), precision=PREC)
        + be1_ref[0])
    ef = (lax.dot_general(he, we2_ref[0], (((1,), (1,)), ((), ())),
                          precision=PREC) + be2_ref[0])
    q = (lax.dot_general(ea, a3_ref[0], (((1,), (1,)), ((), ())),
                         precision=PREC) + ba1_ref[0])
    q_ref[0] = _pack_bf16_words(q)
    ef_ref[0, 0] = ef[:, :128]
    ef_ref[0, 1] = ef[:, 128:]


def _edge_dense(ea_p, We1, be1, We2, be2, A3, ba1):
    BE = 2048
    nl = We1.shape[0]
    grid = (nl, E_PAD // BE)
    return pl.pallas_call(
        _edge_dense_body,
        grid=grid,
        in_specs=[
            pl.BlockSpec((BE, ED), lambda l, e: (e, 0)),
            pl.BlockSpec((1, D, ED), lambda l, e: (l, 0, 0)),
            pl.BlockSpec((1, 1, D), lambda l, e: (l, 0, 0)),
            pl.BlockSpec((1, D, D), lambda l, e: (l, 0, 0)),
            pl.BlockSpec((1, 1, D), lambda l, e: (l, 0, 0)),
            pl.BlockSpec((1, D, ED), lambda l, e: (l, 0, 0)),
            pl.BlockSpec((1, 1, D), lambda l, e: (l, 0, 0)),
        ],
        out_specs=[
            pl.BlockSpec((1, BE, 128), lambda l, e: (l, e, 0)),
            pl.BlockSpec((1, 2, BE, 128), lambda l, e: (l, 0, e, 0)),
        ],
        out_shape=[
            jax.ShapeDtypeStruct((nl, E_PAD, 128), jnp.int32),
            jax.ShapeDtypeStruct((nl, 2, E_PAD, 128), F32),
        ],
    )(ea_p, We1, be1, We2, be2, A3, ba1)


def _node_dense_body(h_ref, wn_ref, a1_ref, a2_ref, p1_ref, p2_ref, xl2_ref):
    h = h_ref[...]
    xl = lax.dot_general(h, wn_ref[...], (((1,), (1,)), ((), ())),
                         precision=PREC)
    p1 = lax.dot_general(xl, a1_ref[...], (((1,), (1,)), ((), ())),
                         precision=PREC)
    p2 = lax.dot_general(xl, a2_ref[...], (((1,), (1,)), ((), ())),
                         precision=PREC)
    p1_ref[...] = _pack_bf16_words(p1)
    p2_ref[...] = _pack_bf16_words(p2)
    xl2_ref[0] = xl[:, :128]
    xl2_ref[1] = xl[:, 128:]


def _node_dense(h, Wn, A1, A2):
    BN = 1000
    grid = (N // BN,)
    return pl.pallas_call(
        _node_dense_body,
        grid=grid,
        in_specs=[
            pl.BlockSpec((BN, D), lambda i: (i, 0)),
            pl.BlockSpec((D, D), lambda i: (0, 0)),
            pl.BlockSpec((D, D), lambda i: (0, 0)),
            pl.BlockSpec((D, D), lambda i: (0, 0)),
        ],
        out_specs=[
            pl.BlockSpec((BN, 128), lambda i: (i, 0)),
            pl.BlockSpec((BN, 128), lambda i: (i, 0)),
            pl.BlockSpec((2, BN, 128), lambda i: (0, i, 0)),
        ],
        out_shape=[
            jax.ShapeDtypeStruct((N, 128), jnp.int32),
            jax.ShapeDtypeStruct((N, 128), jnp.int32),
            jax.ShapeDtypeStruct((2, N, 128), F32),
        ],
    )(h, Wn, A1, A2)


def _combine_body(s0_ref, s1_ref, h_ref, cb_ref, g_ref, b_ref, out_ref, *,
                  relu):
    u = jnp.concatenate([s0_ref[...], s1_ref[...]], axis=1) + cb_ref[...]
    m = jnp.mean(u, axis=-1, keepdims=True)
    v = jnp.mean((u - m) ** 2, axis=-1, keepdims=True)
    y = g_ref[...] * (u - m) * lax.rsqrt(v + 1e-5) + b_ref[...]
    if relu:
        y = jnp.maximum(y, 0.0)
    out_ref[...] = y + h_ref[...]


def _combine(s0, s1, h, cb, g, b, relu):
    BN = 1000
    return pl.pallas_call(
        functools.partial(_combine_body, relu=relu),
        grid=(N // BN,),
        in_specs=[
            pl.BlockSpec((BN, 128), lambda i: (i, 0)),
            pl.BlockSpec((BN, 128), lambda i: (i, 0)),
            pl.BlockSpec((BN, D), lambda i: (i, 0)),
            pl.BlockSpec((1, D), lambda i: (0, 0)),
            pl.BlockSpec((1, D), lambda i: (0, 0)),
            pl.BlockSpec((1, D), lambda i: (0, 0)),
        ],
        out_specs=pl.BlockSpec((BN, D), lambda i: (i, 0)),
        out_shape=jax.ShapeDtypeStruct((N, D), F32),
    )(s0, s1, h, cb, g, b)


_MESH = plsc.VectorSubcoreMesh(core_axis_name="c", subcore_axis_name="s",
                               num_cores=NCORES, num_subcores=NSUB)
_SC_PARAMS = pltpu.CompilerParams(needs_layout_passes=False)


def _attn_body(p1_hbm, p2_hbm, q_hbm, src_hbm, dst_hbm, wa2_hbm, ba2_hbm,
               a_hbm, srcall, dstall, p1r, p2r, qr, zbuf, wa2v, ba2v,
               sem0, sem1):
    c = lax.axis_index("c")
    s = lax.axis_index("s")
    w = s * NCORES + c
    ncht = E_PAD // CA // NW
    base0 = w * (ncht * CA)
    sems = [sem0, sem1]

    pltpu.sync_copy(src_hbm.at[pl.ds(base0, ncht * CA)], srcall)
    pltpu.sync_copy(dst_hbm.at[pl.ds(base0, ncht * CA)], dstall)
    pltpu.sync_copy(wa2_hbm, wa2v)
    pltpu.sync_copy(ba2_hbm, ba2v)

    def issue(t, b):
        isl = pl.ds(t * CA, CA)
        pltpu.async_copy(p1_hbm.at[dstall.at[isl]], p1r.at[b], sems[b])
        pltpu.async_copy(p2_hbm.at[srcall.at[isl]], p2r.at[b], sems[b])
        pltpu.async_copy(q_hbm.at[pl.ds(base0 + t * CA, CA)], qr.at[b],
                         sems[b])

    def wait(b):
        isl = pl.ds(0, CA)
        pltpu.make_async_copy(p1_hbm.at[dstall.at[isl]], p1r.at[b],
                              sems[b]).wait()
        pltpu.make_async_copy(p2_hbm.at[srcall.at[isl]], p2r.at[b],
                              sems[b]).wait()
        pltpu.make_async_copy(q_hbm.at[pl.ds(base0, CA)], qr.at[b],
                              sems[b]).wait()

    issue(0, 0)
    wa2_regs = [plsc.bitcast(wa2v[pl.ds(16 * g, 16)], BF16)
                for g in range(8)]
    ba2 = ba2v[...]
    lane0 = lax.iota(jnp.int32, 16) == 0
    zeros16 = jnp.zeros((16,), F32)

    def group_body(g, carry):
        for b in range(2):
            t = g * 2 + b
            wait(b)
            issue(jnp.minimum(t + 1, ncht - 1), 1 - b)

            def edge_body(p, carry2):
                for e in (2 * p, 2 * p + 1):
                    acc0 = zeros16
                    acc1 = zeros16
                    for j in range(8):
                        sl = pl.ds(16 * j, 16)
                        v = (plsc.bitcast(p1r[b, e, sl], BF16)
                             + plsc.bitcast(p2r[b, e, sl], BF16)
                             + plsc.bitcast(qr[b, e, sl], BF16))
                        v = jnp.maximum(v, 0.0)
                        prod = v * wa2_regs[j]
                        u0, u1 = plsc.unpack(
                            prod, format=plsc.PackFormat.INTERLEAVED)
                        acc0 = acc0 + u0
                        acc1 = acc1 + u1
                    zvec = jnp.broadcast_to(jnp.sum(acc0 + acc1), (16,))
                    eidx = jnp.broadcast_to(e.astype(jnp.int32), (16,))
                    plsc.store_scatter(zbuf, [eidx], zvec, mask=lane0)
                return carry2

            lax.fori_loop(0, CA // 2, edge_body, 0)
            for i in range(CA // 16):
                sl = pl.ds(16 * i, 16)
                z = zbuf[sl] + ba2
                zbuf[sl] = 1.0 / (1.0 + jnp.exp(-z))
            pltpu.sync_copy(zbuf, a_hbm.at[pl.ds(base0 + t * CA, CA)])
        return carry

    lax.fori_loop(0, ncht // 2, group_body, 0)
    wait(0)


def _attn(p1b, p2b, q_l, src_p, dst_p, wa2, ba2v):
    f = pl.kernel(
        _attn_body,
        out_type=jax.ShapeDtypeStruct((E_PAD,), F32),
        mesh=_MESH,
        compiler_params=_SC_PARAMS,
        scratch_types=[
            pltpu.VMEM((E_PAD // NW,), jnp.int32),
            pltpu.VMEM((E_PAD // NW,), jnp.int32),
            pltpu.VMEM((2, CA, 128), jnp.int32),
            pltpu.VMEM((2, CA, 128), jnp.int32),
            pltpu.VMEM((2, CA, 128), jnp.int32),
            pltpu.VMEM((CA,), F32),
            pltpu.VMEM((128,), jnp.int32),
            pltpu.VMEM((16,), F32),
            pltpu.SemaphoreType.DMA,
            pltpu.SemaphoreType.DMA,
        ],
    )
    return f(p1b, p2b, q_l, src_p, dst_p, wa2, ba2v)


def _agg_body(xl2_hbm, ef2_hbm, a_hbm, src_hbm, dst_hbm,
              s_hbm, src2all, dstv, av, xlr, efr, acc,
              sem0, sem1, scs0, scs1):
    c = lax.axis_index("c")
    s = lax.axis_index("s")
    ncht = E_PAD // CB // NSUB
    base0 = s * (ncht * CB)
    sems = [sem0, sem1]
    scs = [scs0, scs1]

    pltpu.sync_copy(src_hbm.at[pl.ds(base0, ncht * CB)], src2all)
    off = c * N

    def addoff(i, carry):
        sl = pl.ds(16 * i, 16)
        src2all[sl] = src2all[sl] + off
        return carry

    lax.fori_loop(0, ncht * CB // 16, addoff, 0)

    zeros16 = jnp.zeros((16,), F32)

    def zrow(r, carry):
        for j in range(8):
            xlr[0, r, pl.ds(16 * j, 16)] = zeros16
        return carry

    lax.fori_loop(0, CB, zrow, 0)
    r0 = s * ROWS_PER_TILE
    for j in range(ROWS_PER_TILE // CB):
        pltpu.sync_copy(xlr.at[0], acc.at[pl.ds(r0 + j * CB, CB)])
    plsc.subcore_barrier()

    def issue(t, b):
        pltpu.async_copy(xl2_hbm.at[src2all.at[pl.ds(t * CB, CB)]],
                         xlr.at[b], sems[b])
        pltpu.async_copy(ef2_hbm.at[pl.ds(c * E_PAD + base0 + t * CB, CB)],
                         efr.at[b], sems[b])
        pltpu.async_copy(dst_hbm.at[pl.ds(base0 + t * CB, CB)],
                         dstv.at[b], sems[b])
        pltpu.async_copy(a_hbm.at[pl.ds(base0 + t * CB, CB)],
                         av.at[b, pl.ds(0, CB)], sems[b])

    def wait(b):
        pltpu.make_async_copy(xl2_hbm.at[src2all.at[pl.ds(0, CB)]],
                              xlr.at[b], sems[b]).wait()
        pltpu.make_async_copy(ef2_hbm.at[pl.ds(0, CB)], efr.at[b],
                              sems[b]).wait()
        pltpu.make_async_copy(dst_hbm.at[pl.ds(0, CB)], dstv.at[b],
                              sems[b]).wait()
        pltpu.make_async_copy(a_hbm.at[pl.ds(0, CB)],
                              av.at[b, pl.ds(0, CB)], sems[b]).wait()

    def wait_scatter(b):
        pltpu.make_async_copy(xlr.at[b], acc.at[dstv.at[b]], scs[b]).wait()

    issue(0, 0)

    def group_body(g, carry):
        for b in range(2):
            t = g * 2 + b
            wait(b)
            if b == 0:
                @pl.when(g >= 1)
                def _():
                    wait_scatter(1)
            else:
                wait_scatter(0)
            issue(jnp.minimum(t + 1, ncht - 1), 1 - b)

            def pair_body(p, carry2):
                e0 = 2 * p
                ae0 = jnp.broadcast_to(av[b, pl.ds(e0, 16)][0], (16,))
                ae1 = jnp.broadcast_to(av[b, pl.ds(e0 + 1, 16)][0], (16,))
                for j in range(8):
                    sl = pl.ds(16 * j, 16)
                    xlr[b, e0, sl] = (xlr[b, e0, sl] + efr[b, e0, sl]) * ae0
                    xlr[b, e0 + 1, sl] = (
                        xlr[b, e0 + 1, sl] + efr[b, e0 + 1, sl]) * ae1
                return carry2

            lax.fori_loop(0, CB // 2, pair_body, 0)
            pltpu.async_copy(xlr.at[b], acc.at[dstv.at[b]], scs[b],
                             add=True)
        return carry

    lax.fori_loop(0, ncht // 2, group_body, 0)
    wait(0)
    wait_scatter(1)
    plsc.subcore_barrier()

    for j in range(ROWS_PER_TILE // 128):
        pltpu.sync_copy(acc.at[pl.ds(r0 + j * 128, 128)],
                        s_hbm.at[pl.ds(c * N_ACC + r0 + j * 128, 128)])


def _aggregate(xl2f, ef2f, a, src_p, dst_p):
    f = pl.kernel(
        _agg_body,
        out_type=jax.ShapeDtypeStruct((NCORES * N_ACC, 128), F32),
        mesh=_MESH,
        compiler_params=_SC_PARAMS,
        scratch_types=[
            pltpu.VMEM((E_PAD // NSUB,), jnp.int32),
            pltpu.VMEM((2, CB), jnp.int32),
            pltpu.VMEM((2, CB + 16), F32),
            pltpu.VMEM((2, CB, 128), F32),
            pltpu.VMEM((2, CB, 128), F32),
            pltpu.VMEM_SHARED((N_ACC, 128), F32),
            pltpu.SemaphoreType.DMA,
            pltpu.SemaphoreType.DMA,
            pltpu.SemaphoreType.DMA,
            pltpu.SemaphoreType.DMA,
        ],
    )
    return f(xl2f, ef2f, a, src_p, dst_p)


def kernel(x, edge_index, edge_attr, W_node, W_e1, b_e1, W_e2, b_e2,
           W_a1, b_a1, W_a2, b_a2, conv_bias, ln_g, ln_b):
    src = edge_index[0].astype(jnp.int32)
    dst = edge_index[1].astype(jnp.int32)
    pad = E_PAD - E
    src_p = jnp.concatenate([src, jnp.zeros((pad,), jnp.int32)])
    dst_p = jnp.concatenate([dst, jnp.full((pad,), N, jnp.int32)])
    ea_p = jnp.pad(edge_attr, ((0, pad), (0, 0)))

    A1 = W_a1[:, :, :D]
    A2 = W_a1[:, :, D:2 * D]
    A3 = W_a1[:, :, 2 * D:]

    h = x
    for i in range(L):
        q_i, ef_i = _edge_dense(ea_p, W_e1[i:i + 1],
                                b_e1[i:i + 1].reshape(1, 1, D),
                                W_e2[i:i + 1], b_e2[i:i + 1].reshape(1, 1, D),
                                A3[i:i + 1], b_a1[i:i + 1].reshape(1, 1, D))
        p1b, p2b, xl2 = _node_dense(h, W_node[i], A1[i], A2[i])
        xl2f = xl2.reshape(2 * N, 128)
        wa2 = _pack_bf16_words(W_a2[i, 0])
        ba2v = jnp.full((16,), b_a2[i, 0], F32)
        a = _attn(p1b, p2b, q_i[0], src_p, dst_p, wa2, ba2v)
        ef2f = ef_i[0].reshape(2 * E_PAD, 128)
        sflat = _aggregate(xl2f, ef2f, a, src_p, dst_p)
        s_split = sflat.reshape(2, N_ACC, 128)
        h = _combine(s_split[0, :N], s_split[1, :N], h, conv_bias[i:i + 1],
                     ln_g[i:i + 1], ln_b[i:i + 1], relu=(i < L - 1))
    return h
